# Initial kernel scaffold; baseline (speedup 1.0000x reference)
import jax
import jax.numpy as jnp
from jax.experimental import pallas as pl


def kernel(x):
    raise NotImplementedError("write your pallas kernel here")


# speedup vs baseline: 28.0333x; 28.0333x over previous
import functools

import jax
import jax.numpy as jnp
from jax import lax
from jax.experimental import pallas as pl
from jax.experimental.pallas import tpu as pltpu
from jax.experimental.pallas import tpu_sc as plsc

_B = 8
_HW = 224 * 224
_C = 96
_L = 16
_NGROUPS = _C // _L
_NUNITS = _B * _NGROUPS
_NTILES = 32
_CHUNK = 1792
_NCHUNKS = _HW // _CHUNK
_UNROLL = 8


def _topk_body(x_hbm, out_hbm, buf0, buf1, res, sem0, sem1):
    cid = lax.axis_index("c")
    sid = lax.axis_index("s")
    wid = sid * 2 + cid

    def process_unit(u):
        b = u // _NGROUPS
        g = u % _NGROUPS
        row0 = b * _HW
        col0 = g * _L

        def start_dma(ci, buf, sem):
            return pltpu.async_copy(
                x_hbm.at[pl.ds(row0 + ci * _CHUNK, _CHUNK), pl.ds(col0, _L)],
                buf, sem)

        def merge_rows(buf, ms):
            def row_body(ri, ms):
                m0, m1, m2, m3, m4 = ms
                base = ri * _UNROLL
                for k in range(_UNROLL):
                    v = buf[base + k, :]
                    hi = jnp.maximum(m0, v); v = jnp.minimum(m0, v); m0 = hi
                    hi = jnp.maximum(m1, v); v = jnp.minimum(m1, v); m1 = hi
                    hi = jnp.maximum(m2, v); v = jnp.minimum(m2, v); m2 = hi
                    hi = jnp.maximum(m3, v); v = jnp.minimum(m3, v); m3 = hi
                    m4 = jnp.maximum(m4, v)
                return (m0, m1, m2, m3, m4)
            return lax.fori_loop(0, _CHUNK // _UNROLL, row_body, ms)

        start_dma(0, buf0, sem0)
        start_dma(1, buf1, sem1)

        def wait_dma(buf, sem):
            pltpu.make_async_copy(
                x_hbm.at[pl.ds(row0, _CHUNK), pl.ds(col0, _L)], buf, sem
            ).wait()

        def pair_body(j, ms):
            wait_dma(buf0, sem0)
            ms = merge_rows(buf0, ms)

            @pl.when(j < _NCHUNKS // 2 - 1)
            def _():
                start_dma(2 * j + 2, buf0, sem0)

            wait_dma(buf1, sem1)
            ms = merge_rows(buf1, ms)

            @pl.when(j < _NCHUNKS // 2 - 1)
            def _():
                start_dma(2 * j + 3, buf1, sem1)

            return ms

        neg = jnp.full((_L,), -jnp.inf, jnp.float32)
        ms = lax.fori_loop(0, _NCHUNKS // 2, pair_body,
                           (neg, neg, neg, neg, neg))
        res[...] = (ms[0] + ms[1] + ms[2] + ms[3] + ms[4]) * jnp.float32(0.2)
        pltpu.sync_copy(res, out_hbm.at[b, pl.ds(col0, _L)])

    process_unit(wid)

    @pl.when(wid < _NUNITS - _NTILES)
    def _():
        process_unit(wid + _NTILES)


@jax.jit
def _run(x2d):
    mesh = plsc.VectorSubcoreMesh(core_axis_name="c", subcore_axis_name="s")
    f = functools.partial(
        pl.kernel,
        mesh=mesh,
        out_type=jax.ShapeDtypeStruct((_B, _C), jnp.float32),
        scratch_types=[
            pltpu.VMEM((_CHUNK, _L), jnp.float32),
            pltpu.VMEM((_CHUNK, _L), jnp.float32),
            pltpu.VMEM((_L,), jnp.float32),
            pltpu.SemaphoreType.DMA,
            pltpu.SemaphoreType.DMA,
        ],
        compiler_params=pltpu.CompilerParams(use_tc_tiling_on_sc=False),
    )(_topk_body)
    return f(x2d)


def kernel(x):
    return _run(jnp.reshape(x, (_B * _HW, _C)))
